# baseline (speedup 1.0000x reference)
import jax
from jax.experimental import pallas as pl
from jax.experimental.pallas import tpu as pltpu

_S_CHUNK = 2048


def _add_pos_kernel(x_ref, pos_ref, scale_ref, out_ref):
    out_ref[...] = x_ref[...] + pos_ref[...] * scale_ref[0]


def kernel(x, pos_embedding, scale):
    batch, seq_len, dim = x.shape
    num_chunks = seq_len // _S_CHUNK
    pos = pos_embedding[:seq_len]
    return pl.pallas_call(
        _add_pos_kernel,
        grid=(num_chunks, batch),
        in_specs=[
            pl.BlockSpec((1, _S_CHUNK, dim), lambda i, j: (j, i, 0)),
            pl.BlockSpec((_S_CHUNK, dim), lambda i, j: (i, 0)),
            pl.BlockSpec(memory_space=pltpu.SMEM),
        ],
        out_specs=pl.BlockSpec((1, _S_CHUNK, dim), lambda i, j: (j, i, 0)),
        out_shape=jax.ShapeDtypeStruct(x.shape, x.dtype),
    )(x, pos, scale)
